# baseline (speedup 1.0000x reference)
import functools

import jax
import jax.numpy as jnp
from jax import lax
from jax.experimental import pallas as pl
from jax.experimental.pallas import tpu as pltpu
from jax.experimental.pallas import tpu_sc as plsc

_NC, _NS = 2, 16
_NW = _NC * _NS
_CHUNK = 128
_LANES = 16


def _xalpha_body(xf_ref, amat_ref, out_ref):
    out_ref[...] = jnp.dot(xf_ref[...], amat_ref[...],
                           preferred_element_type=jnp.float32,
                           precision=jax.lax.Precision.HIGHEST)


def _head_body(xa_ref, pp_ref, d0_ref, d1_ref, wself_ref, wneigh_ref,
               bsage_ref, w1_ref, b1_ref, w2_ref, b2_ref, out_ref, hid_ref):
    agg = pp_ref[0] + pp_ref[1]
    deg = d0_ref[...] + d1_ref[...]
    neigh = agg / (deg + 1e-6)
    hmat = (jnp.dot(xa_ref[...], wself_ref[...],
                    preferred_element_type=jnp.float32)
            + jnp.dot(neigh, wneigh_ref[...],
                      preferred_element_type=jnp.float32)
            + bsage_ref[...])
    hid_ref[...] = hmat
    h1 = jnp.dot(jnp.maximum(hmat, 0.0), w1_ref[...],
                 preferred_element_type=jnp.float32) + b1_ref[...]
    out_ref[...] = jnp.dot(jnp.maximum(h1, 0.0), w2_ref[...],
                           preferred_element_type=jnp.float32) + b2_ref[...]


def _make_sc_scatter(n_pad, f, e):
    n_chunks = e // _CHUNK
    nfull, rem = divmod(n_chunks, _NW)
    assert nfull % 2 == 0
    rows_per_tile = n_pad // _NS
    copies = rows_per_tile // _CHUNK
    groups = f // _LANES
    mesh = plsc.VectorSubcoreMesh(core_axis_name="c", subcore_axis_name="s",
                                  num_cores=_NC, num_subcores=_NS)

    @functools.partial(
        pl.kernel,
        out_type=[jax.ShapeDtypeStruct((_NC, n_pad, f), jnp.float32),
                  jax.ShapeDtypeStruct((n_pad,), jnp.float32),
                  jax.ShapeDtypeStruct((n_pad,), jnp.float32)],
        mesh=mesh,
        compiler_params=pltpu.CompilerParams(needs_layout_passes=False,
                                             use_tc_tiling_on_sc=True),
        scratch_types=[
            pltpu.VMEM((_CHUNK,), jnp.int32),
            pltpu.VMEM((_CHUNK,), jnp.int32),
            pltpu.VMEM((_CHUNK,), jnp.int32),
            pltpu.VMEM((_CHUNK,), jnp.int32),
            pltpu.VMEM((_CHUNK,), jnp.float32),
            pltpu.VMEM((_CHUNK,), jnp.float32),
            pltpu.VMEM((_CHUNK, f), jnp.float32),
            pltpu.VMEM((_CHUNK, f), jnp.float32),
            pltpu.VMEM((n_pad,), jnp.float32),
            pltpu.VMEM((n_pad // 8,), jnp.float32),
            pltpu.VMEM_SHARED((n_pad, f), jnp.float32),
            pltpu.VMEM_SHARED((_NS * (n_pad // 8),), jnp.float32),
            pltpu.SemaphoreType.DMA,
            pltpu.SemaphoreType.DMA,
            pltpu.SemaphoreType.DMA,
            pltpu.SemaphoreType.DMA,
        ],
    )
    def sc_kernel(xa, src, dst, attr, outp, outd0, outd1,
                  srcv0, srcv1, dstv0, dstv1, attrv0, attrv1, rows0, rows1,
                  degv, redbuf, agg_sh, degstage_sh,
                  gsem0, gsem1, isem0, isem1):
        cid = lax.axis_index("c")
        sid = lax.axis_index("s")
        wid = cid * _NS + sid
        lane_iota = lax.iota(jnp.int32, _LANES)
        buf_a = (srcv0, dstv0, attrv0, rows0, gsem0, isem0)
        buf_b = (srcv1, dstv1, attrv1, rows1, gsem1, isem1)

        def zero_row(i, carry):
            for g in range(groups):
                rows0[i, pl.ds(g * _LANES, _LANES)] = jnp.zeros(
                    (_LANES,), jnp.float32)
            return carry
        lax.fori_loop(0, _CHUNK, zero_row, 0)

        def zero_deg(i, carry):
            degv[pl.ds(i * _LANES, _LANES)] = jnp.zeros((_LANES,),
                                                        jnp.float32)
            return carry
        lax.fori_loop(0, n_pad // _LANES, zero_deg, 0)

        row0 = sid * rows_per_tile
        for r in range(copies):
            pltpu.sync_copy(rows0, agg_sh.at[pl.ds(row0 + r * _CHUNK,
                                                   _CHUNK)])
        plsc.subcore_barrier()

        def chunk_base(k):
            return (k * _NW + wid) * _CHUNK

        def idx_slices(k, buf):
            base = chunk_base(k)
            return ((src.at[pl.ds(base, _CHUNK)], buf[0]),
                    (dst.at[pl.ds(base, _CHUNK)], buf[1]),
                    (attr.at[pl.ds(base, _CHUNK)], buf[2]))

        def load_idx_async(k, buf):
            for s_ref, d_ref in idx_slices(k, buf):
                pltpu.async_copy(s_ref, d_ref, buf[5])

        def wait_idx(k, buf):
            for s_ref, d_ref in idx_slices(k, buf):
                pltpu.make_async_copy(s_ref, d_ref, buf[5]).wait()

        def scale_scatter(buf):
            _, dstv, attrv, rows = buf[0], buf[1], buf[2], buf[3]

            def scale_group(g2, c2):
                a16 = attrv[pl.ds(g2 * _LANES, _LANES)]
                d16 = dstv[pl.ds(g2 * _LANES, _LANES)]
                for j in range(_LANES):
                    ab = a16.at[jnp.full((_LANES,), j, jnp.int32)].get(
                        mode="promise_in_bounds")
                    i = g2 * _LANES + j
                    for g in range(groups):
                        sl = pl.ds(g * _LANES, _LANES)
                        rows[i, sl] = rows[i, sl] * ab
                    plsc.addupdate_scatter(degv, [d16], a16,
                                           mask=lane_iota == j)
                return c2
            lax.fori_loop(0, _CHUNK // _LANES, scale_group, 0)
            pltpu.sync_copy(rows, agg_sh.at[dstv], add=True)

        def phase(k, cur, nxt):
            @pl.when(k + 1 < nfull)
            def _():
                wait_idx(k + 1, nxt)
                pltpu.async_copy(xa.at[nxt[0]], nxt[3], nxt[4])
            pltpu.make_async_copy(xa.at[cur[0]], cur[3], cur[4]).wait()
            scale_scatter(cur)

            @pl.when(k + 2 < nfull)
            def _():
                load_idx_async(k + 2, cur)

        for s_ref, d_ref in idx_slices(0, buf_a):
            pltpu.sync_copy(s_ref, d_ref)
        pltpu.async_copy(xa.at[buf_a[0]], buf_a[3], buf_a[4])
        load_idx_async(1, buf_b)

        def pair_body(kk, carry):
            phase(2 * kk, buf_a, buf_b)
            phase(2 * kk + 1, buf_b, buf_a)
            return carry
        lax.fori_loop(0, nfull // 2, pair_body, 0)

        if rem:
            @pl.when(wid < rem)
            def _():
                k_tail = nfull * _NW + wid
                base = pl.multiple_of(k_tail * _CHUNK, _CHUNK)
                pltpu.sync_copy(src.at[pl.ds(base, _CHUNK)], srcv0)
                pltpu.sync_copy(dst.at[pl.ds(base, _CHUNK)], dstv0)
                pltpu.sync_copy(attr.at[pl.ds(base, _CHUNK)], attrv0)
                pltpu.async_copy(xa.at[srcv0], rows0, gsem0).wait()
                scale_scatter(buf_a)

        sec = n_pad // 8
        sub = sec // _NS
        stage0 = pl.multiple_of(sid * sec, 128)
        own0 = pl.multiple_of(sid * sub, 16)
        for q in range(8):
            pltpu.sync_copy(degv.at[pl.ds(q * sec, sec)],
                            degstage_sh.at[pl.ds(stage0, sec)])
            plsc.subcore_barrier()
            for r in range(_NS):
                pltpu.sync_copy(
                    degstage_sh.at[pl.ds(r * sec + own0, sub)],
                    redbuf.at[pl.ds(r * sub, sub)])

            def red_body2(i, carry):
                acc = redbuf[pl.ds(i * _LANES, _LANES)]
                for r in range(1, _NS):
                    acc = acc + redbuf[pl.ds(r * sub + i * _LANES, _LANES)]
                degv[pl.ds(q * sec + own0 + i * _LANES, _LANES)] = acc
                return carry
            lax.fori_loop(0, sub // _LANES, red_body2, 0)
            piece = pl.ds(q * sec + own0, sub)

            @pl.when(cid == 0)
            def _():
                pltpu.sync_copy(degv.at[piece], outd0.at[piece])

            @pl.when(cid == 1)
            def _():
                pltpu.sync_copy(degv.at[piece], outd1.at[piece])
            plsc.subcore_barrier()

        plsc.subcore_barrier()
        for r in range(copies):
            sl = pl.ds(row0 + r * _CHUNK, _CHUNK)
            pltpu.sync_copy(agg_sh.at[sl], outp.at[cid, sl])

    return sc_kernel


def kernel(x, edge_index, edge_attr, W_self, W_neigh, b_sage, att, W1, b1,
           W2, b2):
    n, f, t = x.shape
    e = edge_attr.shape[0]
    hs = W_self.shape[1]
    hid = W1.shape[1]
    od = W2.shape[1]
    stripe = _NS * _CHUNK
    n_pad = ((n + stripe - 1) // stripe) * stripe
    bn = 1000
    assert n % bn == 0 and f % _LANES == 0

    alpha = jax.nn.softmax(att.astype(jnp.float32))
    amat = (jnp.eye(f, dtype=jnp.float32)[:, None, :]
            * alpha[None, :, None]).reshape(f * t, f)
    x_flat = x.reshape(n, f * t)
    assert e % _CHUNK == 0
    src = edge_index[0].astype(jnp.int32)
    dst = edge_index[1].astype(jnp.int32)
    attr = edge_attr.astype(jnp.float32)

    x_alpha = pl.pallas_call(
        _xalpha_body,
        grid=(n // bn,),
        in_specs=[pl.BlockSpec((bn, f * t), lambda i: (i, 0)),
                  pl.BlockSpec((f * t, f), lambda i: (0, 0))],
        out_specs=pl.BlockSpec((bn, f), lambda i: (i, 0)),
        out_shape=jax.ShapeDtypeStruct((n, f), jnp.float32),
    )(x_flat, amat)

    partials, deg0, deg1 = _make_sc_scatter(n_pad, f, e)(
        x_alpha, src, dst, attr)
    deg0 = deg0.reshape(n_pad, 1)
    deg1 = deg1.reshape(n_pad, 1)

    out, hidden = pl.pallas_call(
        _head_body,
        grid=(n // bn,),
        in_specs=[
            pl.BlockSpec((bn, f), lambda i: (i, 0)),
            pl.BlockSpec((_NC, bn, f), lambda i: (0, i, 0)),
            pl.BlockSpec((bn, 1), lambda i: (i, 0)),
            pl.BlockSpec((bn, 1), lambda i: (i, 0)),
            pl.BlockSpec((f, hs), lambda i: (0, 0)),
            pl.BlockSpec((f, hs), lambda i: (0, 0)),
            pl.BlockSpec((1, hs), lambda i: (0, 0)),
            pl.BlockSpec((hs, hid), lambda i: (0, 0)),
            pl.BlockSpec((1, hid), lambda i: (0, 0)),
            pl.BlockSpec((hid, od), lambda i: (0, 0)),
            pl.BlockSpec((1, od), lambda i: (0, 0)),
        ],
        out_specs=[pl.BlockSpec((bn, od), lambda i: (i, 0)),
                   pl.BlockSpec((bn, hs), lambda i: (i, 0))],
        out_shape=[jax.ShapeDtypeStruct((n, od), jnp.float32),
                   jax.ShapeDtypeStruct((n, hs), jnp.float32)],
    )(x_alpha, partials, deg0, deg1, W_self, W_neigh,
      b_sage.reshape(1, hs), W1, b1.reshape(1, hid), W2, b2.reshape(1, od))
    return (out, hidden)


# speedup vs baseline: 1.0007x; 1.0007x over previous
import functools

import jax
import jax.numpy as jnp
from jax import lax
from jax.experimental import pallas as pl
from jax.experimental.pallas import tpu as pltpu
from jax.experimental.pallas import tpu_sc as plsc

_NC, _NS = 2, 16
_NW = _NC * _NS
_CHUNK = 128
_LANES = 16


def _xalpha_body(xf_ref, amat_ref, out_ref):
    out_ref[...] = jnp.dot(xf_ref[...], amat_ref[...],
                           preferred_element_type=jnp.float32,
                           precision=jax.lax.Precision.HIGHEST)


def _head_body(xa_ref, pp_ref, d0_ref, d1_ref, wself_ref, wneigh_ref,
               bsage_ref, w1_ref, b1_ref, w2_ref, b2_ref, out_ref, hid_ref):
    agg = pp_ref[0] + pp_ref[1]
    deg = d0_ref[...] + d1_ref[...]
    neigh = agg / (deg + 1e-6)
    hmat = (jnp.dot(xa_ref[...], wself_ref[...],
                    preferred_element_type=jnp.float32)
            + jnp.dot(neigh, wneigh_ref[...],
                      preferred_element_type=jnp.float32)
            + bsage_ref[...])
    hid_ref[...] = hmat
    h1 = jnp.dot(jnp.maximum(hmat, 0.0), w1_ref[...],
                 preferred_element_type=jnp.float32) + b1_ref[...]
    out_ref[...] = jnp.dot(jnp.maximum(h1, 0.0), w2_ref[...],
                           preferred_element_type=jnp.float32) + b2_ref[...]


def _make_sc_scatter(n_pad, f, e):
    n_chunks = e // _CHUNK
    nfull, rem = divmod(n_chunks, _NW)
    assert nfull % 2 == 0
    rows_per_tile = n_pad // _NS
    copies = rows_per_tile // _CHUNK
    groups = f // _LANES
    mesh = plsc.VectorSubcoreMesh(core_axis_name="c", subcore_axis_name="s",
                                  num_cores=_NC, num_subcores=_NS)

    @functools.partial(
        pl.kernel,
        out_type=[jax.ShapeDtypeStruct((_NC, n_pad, f), jnp.float32),
                  jax.ShapeDtypeStruct((n_pad,), jnp.float32),
                  jax.ShapeDtypeStruct((n_pad,), jnp.float32)],
        mesh=mesh,
        compiler_params=pltpu.CompilerParams(needs_layout_passes=False),
        scratch_types=[
            pltpu.VMEM((_CHUNK,), jnp.int32),
            pltpu.VMEM((_CHUNK,), jnp.int32),
            pltpu.VMEM((_CHUNK,), jnp.int32),
            pltpu.VMEM((_CHUNK,), jnp.int32),
            pltpu.VMEM((_CHUNK,), jnp.float32),
            pltpu.VMEM((_CHUNK,), jnp.float32),
            pltpu.VMEM((_CHUNK, f), jnp.float32),
            pltpu.VMEM((_CHUNK, f), jnp.float32),
            pltpu.VMEM((n_pad,), jnp.float32),
            pltpu.VMEM((n_pad // 8,), jnp.float32),
            pltpu.VMEM_SHARED((n_pad, f), jnp.float32),
            pltpu.VMEM_SHARED((_NS * (n_pad // 8),), jnp.float32),
            pltpu.SemaphoreType.DMA,
            pltpu.SemaphoreType.DMA,
            pltpu.SemaphoreType.DMA,
            pltpu.SemaphoreType.DMA,
        ],
    )
    def sc_kernel(xa, src, dst, attr, outp, outd0, outd1,
                  srcv0, srcv1, dstv0, dstv1, attrv0, attrv1, rows0, rows1,
                  degv, redbuf, agg_sh, degstage_sh,
                  gsem0, gsem1, isem0, isem1):
        cid = lax.axis_index("c")
        sid = lax.axis_index("s")
        wid = cid * _NS + sid
        lane_iota = lax.iota(jnp.int32, _LANES)
        buf_a = (srcv0, dstv0, attrv0, rows0, gsem0, isem0)
        buf_b = (srcv1, dstv1, attrv1, rows1, gsem1, isem1)

        def zero_row(i, carry):
            for g in range(groups):
                rows0[i, pl.ds(g * _LANES, _LANES)] = jnp.zeros(
                    (_LANES,), jnp.float32)
            return carry
        lax.fori_loop(0, _CHUNK, zero_row, 0)

        def zero_deg(i, carry):
            degv[pl.ds(i * _LANES, _LANES)] = jnp.zeros((_LANES,),
                                                        jnp.float32)
            return carry
        lax.fori_loop(0, n_pad // _LANES, zero_deg, 0)

        row0 = sid * rows_per_tile
        for r in range(copies):
            pltpu.sync_copy(rows0, agg_sh.at[pl.ds(row0 + r * _CHUNK,
                                                   _CHUNK)])
        plsc.subcore_barrier()

        def chunk_base(k):
            return (k * _NW + wid) * _CHUNK

        def idx_slices(k, buf):
            base = chunk_base(k)
            return ((src.at[pl.ds(base, _CHUNK)], buf[0]),
                    (dst.at[pl.ds(base, _CHUNK)], buf[1]),
                    (attr.at[pl.ds(base, _CHUNK)], buf[2]))

        def load_idx_async(k, buf):
            for s_ref, d_ref in idx_slices(k, buf):
                pltpu.async_copy(s_ref, d_ref, buf[5])

        def wait_idx(k, buf):
            for s_ref, d_ref in idx_slices(k, buf):
                pltpu.make_async_copy(s_ref, d_ref, buf[5]).wait()

        def scale_scatter(buf):
            _, dstv, attrv, rows = buf[0], buf[1], buf[2], buf[3]

            def scale_group(g2, c2):
                a16 = attrv[pl.ds(g2 * _LANES, _LANES)]
                d16 = dstv[pl.ds(g2 * _LANES, _LANES)]
                for j in range(_LANES):
                    ab = a16.at[jnp.full((_LANES,), j, jnp.int32)].get(
                        mode="promise_in_bounds")
                    i = g2 * _LANES + j
                    for g in range(groups):
                        sl = pl.ds(g * _LANES, _LANES)
                        rows[i, sl] = rows[i, sl] * ab
                    plsc.addupdate_scatter(degv, [d16], a16,
                                           mask=lane_iota == j)
                return c2
            lax.fori_loop(0, _CHUNK // _LANES, scale_group, 0)
            pltpu.sync_copy(rows, agg_sh.at[dstv], add=True)

        def phase(k, cur, nxt):
            @pl.when(k + 1 < nfull)
            def _():
                wait_idx(k + 1, nxt)
                pltpu.async_copy(xa.at[nxt[0]], nxt[3], nxt[4])
            pltpu.make_async_copy(xa.at[cur[0]], cur[3], cur[4]).wait()
            scale_scatter(cur)

            @pl.when(k + 2 < nfull)
            def _():
                load_idx_async(k + 2, cur)

        for s_ref, d_ref in idx_slices(0, buf_a):
            pltpu.sync_copy(s_ref, d_ref)
        pltpu.async_copy(xa.at[buf_a[0]], buf_a[3], buf_a[4])
        load_idx_async(1, buf_b)

        def pair_body(kk, carry):
            phase(2 * kk, buf_a, buf_b)
            phase(2 * kk + 1, buf_b, buf_a)
            return carry
        lax.fori_loop(0, nfull // 2, pair_body, 0)

        if rem:
            @pl.when(wid < rem)
            def _():
                k_tail = nfull * _NW + wid
                base = pl.multiple_of(k_tail * _CHUNK, _CHUNK)
                pltpu.sync_copy(src.at[pl.ds(base, _CHUNK)], srcv0)
                pltpu.sync_copy(dst.at[pl.ds(base, _CHUNK)], dstv0)
                pltpu.sync_copy(attr.at[pl.ds(base, _CHUNK)], attrv0)
                pltpu.async_copy(xa.at[srcv0], rows0, gsem0).wait()
                scale_scatter(buf_a)

        sec = n_pad // 8
        sub = sec // _NS
        stage0 = pl.multiple_of(sid * sec, 128)
        own0 = pl.multiple_of(sid * sub, 16)
        for q in range(8):
            pltpu.sync_copy(degv.at[pl.ds(q * sec, sec)],
                            degstage_sh.at[pl.ds(stage0, sec)])
            plsc.subcore_barrier()
            for r in range(_NS):
                pltpu.sync_copy(
                    degstage_sh.at[pl.ds(r * sec + own0, sub)],
                    redbuf.at[pl.ds(r * sub, sub)])

            def red_body2(i, carry):
                acc = redbuf[pl.ds(i * _LANES, _LANES)]
                for r in range(1, _NS):
                    acc = acc + redbuf[pl.ds(r * sub + i * _LANES, _LANES)]
                degv[pl.ds(q * sec + own0 + i * _LANES, _LANES)] = acc
                return carry
            lax.fori_loop(0, sub // _LANES, red_body2, 0)
            piece = pl.ds(q * sec + own0, sub)

            @pl.when(cid == 0)
            def _():
                pltpu.sync_copy(degv.at[piece], outd0.at[piece])

            @pl.when(cid == 1)
            def _():
                pltpu.sync_copy(degv.at[piece], outd1.at[piece])
            plsc.subcore_barrier()

        plsc.subcore_barrier()
        for r in range(copies):
            sl = pl.ds(row0 + r * _CHUNK, _CHUNK)
            pltpu.sync_copy(agg_sh.at[sl], outp.at[cid, sl])

    return sc_kernel


def kernel(x, edge_index, edge_attr, W_self, W_neigh, b_sage, att, W1, b1,
           W2, b2):
    n, f, t = x.shape
    e = edge_attr.shape[0]
    hs = W_self.shape[1]
    hid = W1.shape[1]
    od = W2.shape[1]
    stripe = _NS * _CHUNK
    n_pad = ((n + stripe - 1) // stripe) * stripe
    bn = 1000
    assert n % bn == 0 and f % _LANES == 0

    alpha = jax.nn.softmax(att.astype(jnp.float32))
    amat = (jnp.eye(f, dtype=jnp.float32)[:, None, :]
            * alpha[None, :, None]).reshape(f * t, f)
    x_flat = x.reshape(n, f * t)
    assert e % _CHUNK == 0
    src = edge_index[0].astype(jnp.int32)
    dst = edge_index[1].astype(jnp.int32)
    attr = edge_attr.astype(jnp.float32)

    x_alpha = pl.pallas_call(
        _xalpha_body,
        grid=(n // bn,),
        in_specs=[pl.BlockSpec((bn, f * t), lambda i: (i, 0)),
                  pl.BlockSpec((f * t, f), lambda i: (0, 0))],
        out_specs=pl.BlockSpec((bn, f), lambda i: (i, 0)),
        out_shape=jax.ShapeDtypeStruct((n, f), jnp.float32),
    )(x_flat, amat)

    partials, deg0, deg1 = _make_sc_scatter(n_pad, f, e)(
        x_alpha, src, dst, attr)
    deg0 = deg0.reshape(n_pad, 1)
    deg1 = deg1.reshape(n_pad, 1)

    out, hidden = pl.pallas_call(
        _head_body,
        grid=(n // bn,),
        in_specs=[
            pl.BlockSpec((bn, f), lambda i: (i, 0)),
            pl.BlockSpec((_NC, bn, f), lambda i: (0, i, 0)),
            pl.BlockSpec((bn, 1), lambda i: (i, 0)),
            pl.BlockSpec((bn, 1), lambda i: (i, 0)),
            pl.BlockSpec((f, hs), lambda i: (0, 0)),
            pl.BlockSpec((f, hs), lambda i: (0, 0)),
            pl.BlockSpec((1, hs), lambda i: (0, 0)),
            pl.BlockSpec((hs, hid), lambda i: (0, 0)),
            pl.BlockSpec((1, hid), lambda i: (0, 0)),
            pl.BlockSpec((hid, od), lambda i: (0, 0)),
            pl.BlockSpec((1, od), lambda i: (0, 0)),
        ],
        out_specs=[pl.BlockSpec((bn, od), lambda i: (i, 0)),
                   pl.BlockSpec((bn, hs), lambda i: (i, 0))],
        out_shape=[jax.ShapeDtypeStruct((n, od), jnp.float32),
                   jax.ShapeDtypeStruct((n, hs), jnp.float32)],
    )(x_alpha, partials, deg0, deg1, W_self, W_neigh,
      b_sage.reshape(1, hs), W1, b1.reshape(1, hid), W2, b2.reshape(1, od))
    return (out, hidden)


# speedup vs baseline: 1.0686x; 1.0679x over previous
import functools

import jax
import jax.numpy as jnp
from jax import lax
from jax.experimental import pallas as pl
from jax.experimental.pallas import tpu as pltpu
from jax.experimental.pallas import tpu_sc as plsc

_NC, _NS = 2, 16
_NW = _NC * _NS
_CHUNK = 128
_LANES = 16


def _xalpha_body(xf_ref, amat_ref, out_ref):
    out_ref[...] = jnp.dot(xf_ref[...], amat_ref[...],
                           preferred_element_type=jnp.float32)


def _head_body(xa_ref, pp_ref, d0_ref, d1_ref, wself_ref, wneigh_ref,
               bsage_ref, w1_ref, b1_ref, w2_ref, b2_ref, out_ref, hid_ref):
    agg = pp_ref[0] + pp_ref[1]
    deg = d0_ref[...] + d1_ref[...]
    neigh = agg / (deg + 1e-6)
    hmat = (jnp.dot(xa_ref[...], wself_ref[...],
                    preferred_element_type=jnp.float32)
            + jnp.dot(neigh, wneigh_ref[...],
                      preferred_element_type=jnp.float32)
            + bsage_ref[...])
    hid_ref[...] = hmat
    h1 = jnp.dot(jnp.maximum(hmat, 0.0), w1_ref[...],
                 preferred_element_type=jnp.float32) + b1_ref[...]
    out_ref[...] = jnp.dot(jnp.maximum(h1, 0.0), w2_ref[...],
                           preferred_element_type=jnp.float32) + b2_ref[...]


def _make_sc_scatter(n_pad, f, e):
    n_chunks = e // _CHUNK
    nfull, rem = divmod(n_chunks, _NW)
    assert nfull % 2 == 0
    rows_per_tile = n_pad // _NS
    copies = rows_per_tile // _CHUNK
    groups = f // _LANES
    mesh = plsc.VectorSubcoreMesh(core_axis_name="c", subcore_axis_name="s",
                                  num_cores=_NC, num_subcores=_NS)

    @functools.partial(
        pl.kernel,
        out_type=[jax.ShapeDtypeStruct((_NC, n_pad, f), jnp.float32),
                  jax.ShapeDtypeStruct((n_pad,), jnp.float32),
                  jax.ShapeDtypeStruct((n_pad,), jnp.float32)],
        mesh=mesh,
        compiler_params=pltpu.CompilerParams(needs_layout_passes=False),
        scratch_types=[
            pltpu.VMEM((_CHUNK,), jnp.int32),
            pltpu.VMEM((_CHUNK,), jnp.int32),
            pltpu.VMEM((_CHUNK,), jnp.int32),
            pltpu.VMEM((_CHUNK,), jnp.int32),
            pltpu.VMEM((_CHUNK,), jnp.float32),
            pltpu.VMEM((_CHUNK,), jnp.float32),
            pltpu.VMEM((_CHUNK, f), jnp.float32),
            pltpu.VMEM((_CHUNK, f), jnp.float32),
            pltpu.VMEM((n_pad,), jnp.float32),
            pltpu.VMEM((n_pad // 8,), jnp.float32),
            pltpu.VMEM_SHARED((n_pad, f), jnp.float32),
            pltpu.VMEM_SHARED((_NS * (n_pad // 8),), jnp.float32),
            pltpu.SemaphoreType.DMA,
            pltpu.SemaphoreType.DMA,
            pltpu.SemaphoreType.DMA,
            pltpu.SemaphoreType.DMA,
        ],
    )
    def sc_kernel(xa, src, dst, attr, outp, outd0, outd1,
                  srcv0, srcv1, dstv0, dstv1, attrv0, attrv1, rows0, rows1,
                  degv, redbuf, agg_sh, degstage_sh,
                  gsem0, gsem1, isem0, isem1):
        cid = lax.axis_index("c")
        sid = lax.axis_index("s")
        wid = cid * _NS + sid
        lane_iota = lax.iota(jnp.int32, _LANES)
        buf_a = (srcv0, dstv0, attrv0, rows0, gsem0, isem0)
        buf_b = (srcv1, dstv1, attrv1, rows1, gsem1, isem1)

        def zero_row(i, carry):
            for g in range(groups):
                rows0[i, pl.ds(g * _LANES, _LANES)] = jnp.zeros(
                    (_LANES,), jnp.float32)
            return carry
        lax.fori_loop(0, _CHUNK, zero_row, 0)

        def zero_deg(i, carry):
            degv[pl.ds(i * _LANES, _LANES)] = jnp.zeros((_LANES,),
                                                        jnp.float32)
            return carry
        lax.fori_loop(0, n_pad // _LANES, zero_deg, 0)

        row0 = sid * rows_per_tile
        for r in range(copies):
            pltpu.sync_copy(rows0, agg_sh.at[pl.ds(row0 + r * _CHUNK,
                                                   _CHUNK)])
        plsc.subcore_barrier()

        def chunk_base(k):
            return (k * _NW + wid) * _CHUNK

        def idx_slices(k, buf):
            base = chunk_base(k)
            return ((src.at[pl.ds(base, _CHUNK)], buf[0]),
                    (dst.at[pl.ds(base, _CHUNK)], buf[1]),
                    (attr.at[pl.ds(base, _CHUNK)], buf[2]))

        def load_idx_async(k, buf):
            for s_ref, d_ref in idx_slices(k, buf):
                pltpu.async_copy(s_ref, d_ref, buf[5])

        def wait_idx(k, buf):
            for s_ref, d_ref in idx_slices(k, buf):
                pltpu.make_async_copy(s_ref, d_ref, buf[5]).wait()

        def scale_scatter(buf):
            _, dstv, attrv, rows = buf[0], buf[1], buf[2], buf[3]

            def scale_group(g2, c2):
                a16 = attrv[pl.ds(g2 * _LANES, _LANES)]
                d16 = dstv[pl.ds(g2 * _LANES, _LANES)]
                for j in range(_LANES):
                    ab = a16.at[jnp.full((_LANES,), j, jnp.int32)].get(
                        mode="promise_in_bounds")
                    i = g2 * _LANES + j
                    for g in range(groups):
                        sl = pl.ds(g * _LANES, _LANES)
                        rows[i, sl] = rows[i, sl] * ab
                    plsc.addupdate_scatter(degv, [d16], a16,
                                           mask=lane_iota == j)
                return c2
            lax.fori_loop(0, _CHUNK // _LANES, scale_group, 0)
            pltpu.sync_copy(rows, agg_sh.at[dstv], add=True)

        def phase(k, cur, nxt):
            @pl.when(k + 1 < nfull)
            def _():
                wait_idx(k + 1, nxt)
                pltpu.async_copy(xa.at[nxt[0]], nxt[3], nxt[4])
            pltpu.make_async_copy(xa.at[cur[0]], cur[3], cur[4]).wait()
            scale_scatter(cur)

            @pl.when(k + 2 < nfull)
            def _():
                load_idx_async(k + 2, cur)

        for s_ref, d_ref in idx_slices(0, buf_a):
            pltpu.sync_copy(s_ref, d_ref)
        pltpu.async_copy(xa.at[buf_a[0]], buf_a[3], buf_a[4])
        load_idx_async(1, buf_b)

        def pair_body(kk, carry):
            phase(2 * kk, buf_a, buf_b)
            phase(2 * kk + 1, buf_b, buf_a)
            return carry
        lax.fori_loop(0, nfull // 2, pair_body, 0)

        if rem:
            @pl.when(wid < rem)
            def _():
                k_tail = nfull * _NW + wid
                base = pl.multiple_of(k_tail * _CHUNK, _CHUNK)
                pltpu.sync_copy(src.at[pl.ds(base, _CHUNK)], srcv0)
                pltpu.sync_copy(dst.at[pl.ds(base, _CHUNK)], dstv0)
                pltpu.sync_copy(attr.at[pl.ds(base, _CHUNK)], attrv0)
                pltpu.async_copy(xa.at[srcv0], rows0, gsem0).wait()
                scale_scatter(buf_a)

        sec = n_pad // 8
        sub = sec // _NS
        stage0 = pl.multiple_of(sid * sec, 128)
        own0 = pl.multiple_of(sid * sub, 16)
        for q in range(8):
            pltpu.sync_copy(degv.at[pl.ds(q * sec, sec)],
                            degstage_sh.at[pl.ds(stage0, sec)])
            plsc.subcore_barrier()
            for r in range(_NS):
                pltpu.sync_copy(
                    degstage_sh.at[pl.ds(r * sec + own0, sub)],
                    redbuf.at[pl.ds(r * sub, sub)])

            def red_body2(i, carry):
                acc = redbuf[pl.ds(i * _LANES, _LANES)]
                for r in range(1, _NS):
                    acc = acc + redbuf[pl.ds(r * sub + i * _LANES, _LANES)]
                degv[pl.ds(q * sec + own0 + i * _LANES, _LANES)] = acc
                return carry
            lax.fori_loop(0, sub // _LANES, red_body2, 0)
            piece = pl.ds(q * sec + own0, sub)

            @pl.when(cid == 0)
            def _():
                pltpu.sync_copy(degv.at[piece], outd0.at[piece])

            @pl.when(cid == 1)
            def _():
                pltpu.sync_copy(degv.at[piece], outd1.at[piece])
            plsc.subcore_barrier()

        plsc.subcore_barrier()
        for r in range(copies):
            sl = pl.ds(row0 + r * _CHUNK, _CHUNK)
            pltpu.sync_copy(agg_sh.at[sl], outp.at[cid, sl])

    return sc_kernel


def kernel(x, edge_index, edge_attr, W_self, W_neigh, b_sage, att, W1, b1,
           W2, b2):
    n, f, t = x.shape
    e = edge_attr.shape[0]
    hs = W_self.shape[1]
    hid = W1.shape[1]
    od = W2.shape[1]
    stripe = _NS * _CHUNK
    n_pad = ((n + stripe - 1) // stripe) * stripe
    bn = 1000
    assert n % bn == 0 and f % _LANES == 0

    alpha = jax.nn.softmax(att.astype(jnp.float32))
    amat = (jnp.eye(f, dtype=jnp.float32)[:, None, :]
            * alpha[None, :, None]).reshape(f * t, f)
    x_flat = x.reshape(n, f * t)
    assert e % _CHUNK == 0
    src = edge_index[0].astype(jnp.int32)
    dst = edge_index[1].astype(jnp.int32)
    attr = edge_attr.astype(jnp.float32)

    x_alpha = pl.pallas_call(
        _xalpha_body,
        grid=(n // bn,),
        in_specs=[pl.BlockSpec((bn, f * t), lambda i: (i, 0)),
                  pl.BlockSpec((f * t, f), lambda i: (0, 0))],
        out_specs=pl.BlockSpec((bn, f), lambda i: (i, 0)),
        out_shape=jax.ShapeDtypeStruct((n, f), jnp.float32),
    )(x_flat, amat)

    partials, deg0, deg1 = _make_sc_scatter(n_pad, f, e)(
        x_alpha, src, dst, attr)
    deg0 = deg0.reshape(n_pad, 1)
    deg1 = deg1.reshape(n_pad, 1)

    out, hidden = pl.pallas_call(
        _head_body,
        grid=(n // bn,),
        in_specs=[
            pl.BlockSpec((bn, f), lambda i: (i, 0)),
            pl.BlockSpec((_NC, bn, f), lambda i: (0, i, 0)),
            pl.BlockSpec((bn, 1), lambda i: (i, 0)),
            pl.BlockSpec((bn, 1), lambda i: (i, 0)),
            pl.BlockSpec((f, hs), lambda i: (0, 0)),
            pl.BlockSpec((f, hs), lambda i: (0, 0)),
            pl.BlockSpec((1, hs), lambda i: (0, 0)),
            pl.BlockSpec((hs, hid), lambda i: (0, 0)),
            pl.BlockSpec((1, hid), lambda i: (0, 0)),
            pl.BlockSpec((hid, od), lambda i: (0, 0)),
            pl.BlockSpec((1, od), lambda i: (0, 0)),
        ],
        out_specs=[pl.BlockSpec((bn, od), lambda i: (i, 0)),
                   pl.BlockSpec((bn, hs), lambda i: (i, 0))],
        out_shape=[jax.ShapeDtypeStruct((n, od), jnp.float32),
                   jax.ShapeDtypeStruct((n, hs), jnp.float32)],
    )(x_alpha, partials, deg0, deg1, W_self, W_neigh,
      b_sage.reshape(1, hs), W1, b1.reshape(1, hid), W2, b2.reshape(1, od))
    return (out, hidden)
